# baseline (speedup 1.0000x reference)
import functools

import jax
import jax.numpy as jnp
from jax import lax
from jax.experimental import pallas as pl
from jax.experimental.pallas import tpu as pltpu
from jax.experimental.pallas import tpu_sc as plsc

_NC = 2
_NS = 16
_NW = _NC * _NS
_CH = 128


def _sc_mesh():
    return plsc.VectorSubcoreMesh(core_axis_name="c", subcore_axis_name="s")


def _sc_degree(dstp, n_chunks, NP):
    RPS = NP // _NS

    def body(dst_ref, out_ref, dst_v, ones_v, zero_v, tmp_v, acc, *_):
        c = lax.axis_index("c")
        s = lax.axis_index("s")
        wid = c * _NS + s
        pltpu.sync_copy(dst_ref.at[wid], dst_v)
        for g in range(_CH // 16):
            ones_v[pl.ds(g * 16, 16)] = jnp.ones((16,), jnp.float32)
            zero_v[pl.ds(g * 16, 16)] = jnp.zeros((16,), jnp.float32)
        row0 = s * RPS
        off = 0
        while off < RPS:
            k = min(_CH, RPS - off)
            pltpu.sync_copy(zero_v.at[pl.ds(0, k)], acc.at[pl.ds(row0 + off, k)])
            off += k
        plsc.subcore_barrier()

        @pl.loop(0, n_chunks)
        def _(j):
            pltpu.sync_copy(ones_v, acc.at[dst_v.at[j]], add=True)

        plsc.subcore_barrier()
        off = 0
        while off < RPS:
            k = min(_CH, RPS - off)
            pltpu.sync_copy(acc.at[pl.ds(row0 + off, k)], tmp_v.at[pl.ds(0, k)])
            pltpu.sync_copy(tmp_v.at[pl.ds(0, k)],
                            out_ref.at[c].at[pl.ds(row0 + off, k)])
            off += k

    return pl.kernel(
        body,
        out_type=jax.ShapeDtypeStruct((_NC, NP), jnp.float32),
        mesh=_sc_mesh(),
        scratch_types=[
            pltpu.VMEM((n_chunks, _CH), jnp.int32),
            pltpu.VMEM((_CH,), jnp.float32),
            pltpu.VMEM((_CH,), jnp.float32),
            pltpu.VMEM((_CH,), jnp.float32),
            pltpu.VMEM_SHARED((NP,), jnp.float32),
        ],
    )(dstp)


def _sc_scatter(table, srcp, dstp, n0, n1, NP, D):
    RPS = NP // _NS

    def body(tab_ref, src_ref, dst_ref, out_ref, src_v, dst_v, gbuf, acc,
             sem0, sem1):
        c = lax.axis_index("c")
        s = lax.axis_index("s")
        wid = c * _NS + s
        nc = n0 if n0 == n1 else jnp.where(c == 0, n0, n1)
        pltpu.sync_copy(src_ref.at[wid], src_v)
        pltpu.sync_copy(dst_ref.at[wid], dst_v)

        @pl.loop(0, _CH)
        def _(r):
            for g in range(D // 16):
                gbuf[0, r, pl.ds(g * 16, 16)] = jnp.zeros((16,), jnp.float32)

        row0 = s * RPS
        off = 0
        while off < RPS:
            k = min(_CH, RPS - off)
            pltpu.sync_copy(gbuf.at[0].at[pl.ds(0, k)],
                            acc.at[pl.ds(row0 + off, k)])
            off += k
        plsc.subcore_barrier()

        pltpu.async_copy(tab_ref.at[src_v.at[0]], gbuf.at[0], sem0)
        pltpu.async_copy(tab_ref.at[src_v.at[1]], gbuf.at[1], sem1)

        @pl.loop(0, nc, step=2)
        def _(j):
            pltpu.make_async_copy(tab_ref.at[src_v.at[j]], gbuf.at[0],
                                  sem0).wait()
            pltpu.sync_copy(gbuf.at[0], acc.at[dst_v.at[j]], add=True)

            @pl.when(j + 2 < nc)
            def _():
                pltpu.async_copy(tab_ref.at[src_v.at[j + 2]], gbuf.at[0], sem0)

            pltpu.make_async_copy(tab_ref.at[src_v.at[j + 1]], gbuf.at[1],
                                  sem1).wait()
            pltpu.sync_copy(gbuf.at[1], acc.at[dst_v.at[j + 1]], add=True)

            @pl.when(j + 3 < nc)
            def _():
                pltpu.async_copy(tab_ref.at[src_v.at[j + 3]], gbuf.at[1], sem1)

        plsc.subcore_barrier()
        off = 0
        while off < RPS:
            k = min(_CH, RPS - off)
            pltpu.sync_copy(acc.at[pl.ds(row0 + off, k)],
                            gbuf.at[0].at[pl.ds(0, k)])
            pltpu.sync_copy(gbuf.at[0].at[pl.ds(0, k)],
                            out_ref.at[c].at[pl.ds(row0 + off, k)])
            off += k

    return pl.kernel(
        body,
        out_type=jax.ShapeDtypeStruct((_NC, NP, D), jnp.float32),
        mesh=_sc_mesh(),
        scratch_types=[
            pltpu.VMEM((n0, _CH), jnp.int32),
            pltpu.VMEM((n0, _CH), jnp.int32),
            pltpu.VMEM((2, _CH, D), jnp.float32),
            pltpu.VMEM_SHARED((NP, D), jnp.float32),
            pltpu.SemaphoreType.DMA,
            pltpu.SemaphoreType.DMA,
        ],
    )(table, srcp, dstp)


def _tc_layer1(x, W1, degs, BR):
    N, DIN = x.shape
    H = W1.shape[1]
    NPdim = degs.shape[1]

    def body(x_ref, w_ref, deg_ref, x1p_ref, dinv_ref):
        deg = deg_ref[0] + deg_ref[1] + 1.0
        dinv = lax.rsqrt(jnp.maximum(deg, 1.0))
        h = jnp.dot(x_ref[...], w_ref[...], preferred_element_type=jnp.float32)
        x1p_ref[...] = h * dinv
        dinv_ref[...] = dinv

    return pl.pallas_call(
        body,
        grid=(N // BR,),
        in_specs=[
            pl.BlockSpec((BR, DIN), lambda i: (i, 0)),
            pl.BlockSpec((DIN, H), lambda i: (0, 0)),
            pl.BlockSpec((_NC, BR, 1), lambda i: (0, i, 0)),
        ],
        out_specs=[
            pl.BlockSpec((BR, H), lambda i: (i, 0)),
            pl.BlockSpec((BR, 1), lambda i: (i, 0)),
        ],
        out_shape=[
            jax.ShapeDtypeStruct((N, H), jnp.float32),
            jax.ShapeDtypeStruct((N, 1), jnp.float32),
        ],
    )(x, W1, degs)


def _tc_layer2(acc, x1p, dinv, b1, BR):
    N, H = x1p.shape

    def body(acc_ref, x1p_ref, dinv_ref, b1_ref, out_ref):
        p = (acc_ref[0] + acc_ref[1] + x1p_ref[...]) * dinv_ref[...]
        hrelu = jnp.maximum(p + b1_ref[...], 0.0)
        out_ref[...] = hrelu * dinv_ref[...]

    return pl.pallas_call(
        body,
        grid=(N // BR,),
        in_specs=[
            pl.BlockSpec((_NC, BR, H), lambda i: (0, i, 0)),
            pl.BlockSpec((BR, H), lambda i: (i, 0)),
            pl.BlockSpec((BR, 1), lambda i: (i, 0)),
            pl.BlockSpec((1, H), lambda i: (0, 0)),
        ],
        out_specs=pl.BlockSpec((BR, H), lambda i: (i, 0)),
        out_shape=jax.ShapeDtypeStruct((N, H), jnp.float32),
    )(acc, x1p, dinv, b1)


def _tc_layer3(acc, hp, dinv, W2, b2, BR):
    N, H = hp.shape
    C = W2.shape[1]

    def body(acc_ref, hp_ref, dinv_ref, w2_ref, b2_ref, out_ref):
        pre = (acc_ref[0] + acc_ref[1] + hp_ref[...]) * dinv_ref[...]
        logits = jnp.dot(pre, w2_ref[...], preferred_element_type=jnp.float32)
        logits = logits + b2_ref[...]
        m = jnp.max(logits, axis=1, keepdims=True)
        e = logits - m
        out_ref[...] = e - jnp.log(jnp.sum(jnp.exp(e), axis=1, keepdims=True))

    return pl.pallas_call(
        body,
        grid=(N // BR,),
        in_specs=[
            pl.BlockSpec((_NC, BR, H), lambda i: (0, i, 0)),
            pl.BlockSpec((BR, H), lambda i: (i, 0)),
            pl.BlockSpec((BR, 1), lambda i: (i, 0)),
            pl.BlockSpec((H, C), lambda i: (0, 0)),
            pl.BlockSpec((1, C), lambda i: (0, 0)),
        ],
        out_specs=pl.BlockSpec((BR, C), lambda i: (i, 0)),
        out_shape=jax.ShapeDtypeStruct((N, C), jnp.float32),
    )(acc, hp, dinv, W2, b2)


def kernel(x, edge_index, W1, b1, W2, b2):
    N, DIN = x.shape
    E = edge_index.shape[1]
    H = W1.shape[1]
    C = W2.shape[1]

    n0, n1 = 62, 18
    assert _NS * (n0 + n1) * _CH >= E
    RPS = -(-(N // _NS + 1) // _CH) * _CH
    NP = RPS * _NS

    def pad_vals(k, trash):
        if trash:
            return N + jnp.arange(k, dtype=jnp.int32) % (NP - N)
        return jnp.zeros((k,), jnp.int32)

    def slots(arr, trash):
        e0 = min(_NS * n0 * _CH, E)
        part0 = arr[:e0]
        if _NS * n0 * _CH - e0:
            part0 = jnp.concatenate(
                [part0, pad_vals(_NS * n0 * _CH - e0, trash)])
        part0 = part0.reshape(_NS, n0, _CH)
        rest = arr[e0:]
        p1 = _NS * n1 * _CH - (E - e0)
        if p1:
            rest = jnp.concatenate([rest, pad_vals(p1, trash)])
        part1 = rest.reshape(_NS, n1, _CH)
        if n0 > n1:
            pad_blk = pad_vals(_NS * (n0 - n1) * _CH, trash).reshape(
                _NS, n0 - n1, _CH)
            part1 = jnp.concatenate([part1, pad_blk], axis=1)
        return jnp.concatenate([part0, part1], axis=0)

    srcp = slots(edge_index[0], False)
    dstp = slots(edge_index[1], True)

    BR = 1000
    deg = _sc_degree(dstp, n0, NP)
    x1p, dinv = _tc_layer1(x, W1, deg.reshape(_NC, NP, 1), BR)
    acc1 = _sc_scatter(x1p, srcp, dstp, n0, n1, NP, H)
    hp = _tc_layer2(acc1, x1p, dinv, b1.reshape(1, H), BR)
    acc2 = _sc_scatter(hp, srcp, dstp, n0, n1, NP, H)
    return _tc_layer3(acc2, hp, dinv, W2, b2.reshape(1, C), BR)


# speedup vs baseline: 1.0681x; 1.0681x over previous
import functools

import jax
import jax.numpy as jnp
from jax import lax
from jax.experimental import pallas as pl
from jax.experimental.pallas import tpu as pltpu
from jax.experimental.pallas import tpu_sc as plsc

_NC = 2
_NS = 16
_NW = _NC * _NS
_CH = 128


def _sc_mesh():
    return plsc.VectorSubcoreMesh(core_axis_name="c", subcore_axis_name="s")


def _sc_degree(dstp, n_chunks, NP):
    RPS = NP // _NS

    def body(dst_ref, out_ref, dst_v, ones_v, zero_v, tmp_v, acc, *_):
        c = lax.axis_index("c")
        s = lax.axis_index("s")
        wid = c * _NS + s
        pltpu.sync_copy(dst_ref.at[wid], dst_v)
        for g in range(_CH // 16):
            ones_v[pl.ds(g * 16, 16)] = jnp.ones((16,), jnp.float32)
            zero_v[pl.ds(g * 16, 16)] = jnp.zeros((16,), jnp.float32)
        row0 = s * RPS
        off = 0
        while off < RPS:
            k = min(_CH, RPS - off)
            pltpu.sync_copy(zero_v.at[pl.ds(0, k)], acc.at[pl.ds(row0 + off, k)])
            off += k
        plsc.subcore_barrier()

        @pl.loop(0, n_chunks)
        def _(j):
            pltpu.sync_copy(ones_v, acc.at[dst_v.at[j]], add=True)

        plsc.subcore_barrier()
        off = 0
        while off < RPS:
            k = min(_CH, RPS - off)
            pltpu.sync_copy(acc.at[pl.ds(row0 + off, k)], tmp_v.at[pl.ds(0, k)])
            pltpu.sync_copy(tmp_v.at[pl.ds(0, k)],
                            out_ref.at[c].at[pl.ds(row0 + off, k)])
            off += k

    return pl.kernel(
        body,
        out_type=jax.ShapeDtypeStruct((_NC, NP), jnp.float32),
        mesh=_sc_mesh(),
        scratch_types=[
            pltpu.VMEM((n_chunks, _CH), jnp.int32),
            pltpu.VMEM((_CH,), jnp.float32),
            pltpu.VMEM((_CH,), jnp.float32),
            pltpu.VMEM((_CH,), jnp.float32),
            pltpu.VMEM_SHARED((NP,), jnp.float32),
        ],
    )(dstp)


def _sc_scatter(table, srcp, dstp, n0, n1, NP, D):
    RPS = NP // _NS

    def body(tab_ref, src_ref, dst_ref, out_ref, src_v, dst_v, gbuf, acc,
             sem0, sem1):
        c = lax.axis_index("c")
        s = lax.axis_index("s")
        wid = c * _NS + s
        nc = n0 if n0 == n1 else jnp.where(c == 0, n0, n1)
        pltpu.sync_copy(src_ref.at[wid], src_v)
        pltpu.sync_copy(dst_ref.at[wid], dst_v)

        @pl.loop(0, _CH)
        def _(r):
            for g in range(D // 16):
                gbuf[0, r, pl.ds(g * 16, 16)] = jnp.zeros((16,), jnp.float32)

        row0 = s * RPS
        off = 0
        while off < RPS:
            k = min(_CH, RPS - off)
            pltpu.sync_copy(gbuf.at[0].at[pl.ds(0, k)],
                            acc.at[pl.ds(row0 + off, k)])
            off += k
        plsc.subcore_barrier()

        pltpu.async_copy(tab_ref.at[src_v.at[0]], gbuf.at[0], sem0)
        pltpu.async_copy(tab_ref.at[src_v.at[1]], gbuf.at[1], sem1)

        @pl.loop(0, nc, step=2)
        def _(j):
            pltpu.make_async_copy(tab_ref.at[src_v.at[j]], gbuf.at[0],
                                  sem0).wait()
            pltpu.sync_copy(gbuf.at[0], acc.at[dst_v.at[j]], add=True)

            @pl.when(j + 2 < nc)
            def _():
                pltpu.async_copy(tab_ref.at[src_v.at[j + 2]], gbuf.at[0], sem0)

            pltpu.make_async_copy(tab_ref.at[src_v.at[j + 1]], gbuf.at[1],
                                  sem1).wait()
            pltpu.sync_copy(gbuf.at[1], acc.at[dst_v.at[j + 1]], add=True)

            @pl.when(j + 3 < nc)
            def _():
                pltpu.async_copy(tab_ref.at[src_v.at[j + 3]], gbuf.at[1], sem1)

        plsc.subcore_barrier()
        off = 0
        while off < RPS:
            k = min(_CH, RPS - off)
            pltpu.sync_copy(acc.at[pl.ds(row0 + off, k)],
                            gbuf.at[0].at[pl.ds(0, k)])
            pltpu.sync_copy(gbuf.at[0].at[pl.ds(0, k)],
                            out_ref.at[c].at[pl.ds(row0 + off, k)])
            off += k

    return pl.kernel(
        body,
        out_type=jax.ShapeDtypeStruct((_NC, NP, D), jnp.float32),
        mesh=_sc_mesh(),
        scratch_types=[
            pltpu.VMEM((n0, _CH), jnp.int32),
            pltpu.VMEM((n0, _CH), jnp.int32),
            pltpu.VMEM((2, _CH, D), jnp.float32),
            pltpu.VMEM_SHARED((NP, D), jnp.float32),
            pltpu.SemaphoreType.DMA,
            pltpu.SemaphoreType.DMA,
        ],
    )(table, srcp, dstp)


def _tc_layer1(x, W1, degs, BR):
    N, DIN = x.shape
    H = W1.shape[1]
    NPdim = degs.shape[1]

    def body(x_ref, w_ref, deg_ref, x1p_ref, dinv_ref):
        deg = deg_ref[0] + deg_ref[1] + 1.0
        dinv = lax.rsqrt(jnp.maximum(deg, 1.0))
        h = jnp.dot(x_ref[...], w_ref[...], preferred_element_type=jnp.float32)
        x1p_ref[...] = h * dinv
        dinv_ref[...] = dinv

    return pl.pallas_call(
        body,
        grid=(N // BR,),
        in_specs=[
            pl.BlockSpec((BR, DIN), lambda i: (i, 0)),
            pl.BlockSpec((DIN, H), lambda i: (0, 0)),
            pl.BlockSpec((_NC, BR, 1), lambda i: (0, i, 0)),
        ],
        out_specs=[
            pl.BlockSpec((BR, H), lambda i: (i, 0)),
            pl.BlockSpec((BR, 1), lambda i: (i, 0)),
        ],
        out_shape=[
            jax.ShapeDtypeStruct((N, H), jnp.float32),
            jax.ShapeDtypeStruct((N, 1), jnp.float32),
        ],
    )(x, W1, degs)


def _tc_layer2(acc, x1p, dinv, b1, BR):
    N, H = x1p.shape

    def body(acc_ref, x1p_ref, dinv_ref, b1_ref, out_ref):
        p = (acc_ref[0] + acc_ref[1] + x1p_ref[...]) * dinv_ref[...]
        hrelu = jnp.maximum(p + b1_ref[...], 0.0)
        out_ref[...] = hrelu * dinv_ref[...]

    return pl.pallas_call(
        body,
        grid=(N // BR,),
        in_specs=[
            pl.BlockSpec((_NC, BR, H), lambda i: (0, i, 0)),
            pl.BlockSpec((BR, H), lambda i: (i, 0)),
            pl.BlockSpec((BR, 1), lambda i: (i, 0)),
            pl.BlockSpec((1, H), lambda i: (0, 0)),
        ],
        out_specs=pl.BlockSpec((BR, H), lambda i: (i, 0)),
        out_shape=jax.ShapeDtypeStruct((N, H), jnp.float32),
    )(acc, x1p, dinv, b1)


def _tc_layer3(acc, hp, dinv, W2, b2, BR):
    N, H = hp.shape
    C = W2.shape[1]

    def body(acc_ref, hp_ref, dinv_ref, w2_ref, b2_ref, out_ref):
        pre = (acc_ref[0] + acc_ref[1] + hp_ref[...]) * dinv_ref[...]
        logits = jnp.dot(pre, w2_ref[...], preferred_element_type=jnp.float32)
        logits = logits + b2_ref[...]
        m = jnp.max(logits, axis=1, keepdims=True)
        e = logits - m
        out_ref[...] = e - jnp.log(jnp.sum(jnp.exp(e), axis=1, keepdims=True))

    return pl.pallas_call(
        body,
        grid=(N // BR,),
        in_specs=[
            pl.BlockSpec((_NC, BR, H), lambda i: (0, i, 0)),
            pl.BlockSpec((BR, H), lambda i: (i, 0)),
            pl.BlockSpec((BR, 1), lambda i: (i, 0)),
            pl.BlockSpec((H, C), lambda i: (0, 0)),
            pl.BlockSpec((1, C), lambda i: (0, 0)),
        ],
        out_specs=pl.BlockSpec((BR, C), lambda i: (i, 0)),
        out_shape=jax.ShapeDtypeStruct((N, C), jnp.float32),
    )(acc, hp, dinv, W2, b2)


def kernel(x, edge_index, W1, b1, W2, b2):
    N, DIN = x.shape
    E = edge_index.shape[1]
    H = W1.shape[1]
    C = W2.shape[1]

    n0, n1 = 40, 40
    assert _NS * (n0 + n1) * _CH >= E
    RPS = -(-(N // _NS + 1) // _CH) * _CH
    NP = RPS * _NS

    def pad_vals(k, trash):
        if trash:
            return N + jnp.arange(k, dtype=jnp.int32) % (NP - N)
        return jnp.zeros((k,), jnp.int32)

    def slots(arr, trash):
        e0 = min(_NS * n0 * _CH, E)
        part0 = arr[:e0]
        if _NS * n0 * _CH - e0:
            part0 = jnp.concatenate(
                [part0, pad_vals(_NS * n0 * _CH - e0, trash)])
        part0 = part0.reshape(_NS, n0, _CH)
        rest = arr[e0:]
        p1 = _NS * n1 * _CH - (E - e0)
        if p1:
            rest = jnp.concatenate([rest, pad_vals(p1, trash)])
        part1 = rest.reshape(_NS, n1, _CH)
        if n0 > n1:
            pad_blk = pad_vals(_NS * (n0 - n1) * _CH, trash).reshape(
                _NS, n0 - n1, _CH)
            part1 = jnp.concatenate([part1, pad_blk], axis=1)
        return jnp.concatenate([part0, part1], axis=0)

    srcp = slots(edge_index[0], False)
    dstp = slots(edge_index[1], True)

    BR = 1000
    deg = _sc_degree(dstp, n0, NP)
    x1p, dinv = _tc_layer1(x, W1, deg.reshape(_NC, NP, 1), BR)
    acc1 = _sc_scatter(x1p, srcp, dstp, n0, n1, NP, H)
    hp = _tc_layer2(acc1, x1p, dinv, b1.reshape(1, H), BR)
    acc2 = _sc_scatter(hp, srcp, dstp, n0, n1, NP, H)
    return _tc_layer3(acc2, hp, dinv, W2, b2.reshape(1, C), BR)
